# Initial kernel scaffold; baseline (speedup 1.0000x reference)
import jax
import jax.numpy as jnp
from jax.experimental import pallas as pl


def kernel(hidden_states, Wg, We, be):
    raise NotImplementedError("write your pallas kernel here")


# speedup vs baseline: 1.9254x; 1.9254x over previous
import functools

import jax
import jax.numpy as jnp
from jax import lax
from jax.experimental import pallas as pl
from jax.experimental.pallas import tpu as pltpu
from jax.experimental.pallas import tpu_sc as plsc

E = 64
K = 2
D = 768
N = 4096
CAP = 160
NSLOT = E * CAP
TRASH = NSLOT
PAD_ROW = N

B = 512
NB = N // B

NC = 2
NS = 16
NW = NC * NS
L = 16

ROWS_PER_W = NSLOT // NW
DCH = 40
TPT = N // NW
CCH = 16


def _router_body(x_ref, wg_ref, d0_ref, d1_ref, s0_ref, s1_ref,
                 g0_ref, g1_ref, carry_ref):
    i = pl.program_id(0)

    @pl.when(i == 0)
    def _():
        carry_ref[...] = jnp.zeros_like(carry_ref)

    x = x_ref[...]
    wg = wg_ref[...]
    logits = jnp.dot(x, wg, preferred_element_type=jnp.float32)

    iota = lax.broadcasted_iota(jnp.int32, (B, E), 1)
    m0 = jnp.max(logits, axis=1, keepdims=True)
    idx0 = jnp.min(jnp.where(logits == m0, iota, E), axis=1, keepdims=True)
    oh0 = iota == idx0
    masked = jnp.where(oh0, -jnp.inf, logits)
    m1 = jnp.max(masked, axis=1, keepdims=True)
    idx1 = jnp.min(jnp.where(masked == m1, iota, E), axis=1, keepdims=True)
    oh1 = iota == idx1

    t = jnp.exp(m1 - m0)
    g0 = 1.0 / (1.0 + t)
    g1 = t / (1.0 + t)

    ohsum = jnp.where(oh0 | oh1, 1.0, 0.0)
    r = lax.broadcasted_iota(jnp.int32, (B, B), 0)
    c = lax.broadcasted_iota(jnp.int32, (B, B), 1)
    tril = jnp.where(c < r, 1.0, 0.0)
    cnt_before = (jnp.dot(tril, ohsum, preferred_element_type=jnp.float32)
                  + carry_ref[...])
    carry_ref[...] = carry_ref[...] + jnp.sum(ohsum, axis=0, keepdims=True)

    pos0 = jnp.sum(jnp.where(oh0, cnt_before, 0.0), axis=1)
    pos1 = jnp.sum(jnp.where(oh1, cnt_before, 0.0), axis=1)
    keep0 = pos0 < CAP
    keep1 = pos1 < CAP
    p0c = jnp.minimum(pos0.astype(jnp.int32), CAP - 1)
    p1c = jnp.minimum(pos1.astype(jnp.int32), CAP - 1)
    e0 = idx0[:, 0]
    e1 = idx1[:, 0]
    d0 = e0 * CAP + p0c
    d1 = e1 * CAP + p1c

    d0_ref[...] = d0.reshape(1, 1, B)
    d1_ref[...] = d1.reshape(1, 1, B)
    s0_ref[...] = jnp.where(keep0, d0, TRASH).reshape(1, 1, B)
    s1_ref[...] = jnp.where(keep1, d1, TRASH).reshape(1, 1, B)
    g0_ref[...] = (g0[:, 0] * keep0.astype(jnp.float32)).reshape(1, 1, B)
    g1_ref[...] = (g1[:, 0] * keep1.astype(jnp.float32)).reshape(1, 1, B)


def _router(x, Wg):
    blk = pl.BlockSpec((1, 1, B), lambda i: (i, 0, 0))
    iod = jax.ShapeDtypeStruct((NB, 1, B), jnp.int32)
    fod = jax.ShapeDtypeStruct((NB, 1, B), jnp.float32)
    return pl.pallas_call(
        _router_body,
        grid=(NB,),
        in_specs=[
            pl.BlockSpec((B, D), lambda i: (i, 0)),
            pl.BlockSpec((D, E), lambda i: (0, 0)),
        ],
        out_specs=[blk] * 6,
        out_shape=[iod, iod, iod, iod, fod, fod],
        scratch_shapes=[pltpu.VMEM((1, E), jnp.float32)],
    )(x, Wg)


def _dispatch_body(xpad_hbm, s0_hbm, s1_hbm, disp_hbm,
                   s0_v, s1_v, src_v, buf_v, sem):
    wid = lax.axis_index("s") * NC + lax.axis_index("c")
    base = wid * ROWS_PER_W

    pltpu.sync_copy(s0_hbm, s0_v)
    pltpu.sync_copy(s1_hbm, s1_v)

    def init_body(i, _):
        src_v[pl.ds(i * L, L)] = jnp.full((L,), PAD_ROW, jnp.int32)
        return 0
    lax.fori_loop(0, (NSLOT + L) // L, init_body, 0)

    def scat_body(i, _):
        tok = lax.broadcasted_iota(jnp.int32, (L,), 0) + i * L
        plsc.store_scatter(src_v, [s0_v[pl.ds(i * L, L)]], tok)
        plsc.store_scatter(src_v, [s1_v[pl.ds(i * L, L)]], tok)
        return 0
    lax.fori_loop(0, N // L, scat_body, 0)

    for ch in range(ROWS_PER_W // DCH):
        row0 = base + ch * DCH
        pltpu.async_copy(
            xpad_hbm.at[src_v.at[pl.ds(row0, DCH)]], buf_v, sem).wait()
        pltpu.sync_copy(buf_v, disp_hbm.at[pl.ds(row0, DCH)])


def _dispatch(x_pad, s0, s1):
    mesh = plsc.VectorSubcoreMesh(core_axis_name="c", subcore_axis_name="s")
    f = functools.partial(
        pl.kernel,
        mesh=mesh,
        compiler_params=pltpu.CompilerParams(needs_layout_passes=False),
        out_type=jax.ShapeDtypeStruct((NSLOT, D), jnp.float32),
        scratch_types=[
            pltpu.VMEM((N,), jnp.int32),
            pltpu.VMEM((N,), jnp.int32),
            pltpu.VMEM((NSLOT + L,), jnp.int32),
            pltpu.VMEM((DCH, D), jnp.float32),
            pltpu.SemaphoreType.DMA,
        ],
    )(_dispatch_body)
    return f(x_pad, s0, s1)


def _expert_body(disp_ref, we_ref, be_ref, y_ref):
    a = disp_ref[0]
    w = we_ref[0]
    y_ref[0] = (jnp.dot(a, w, preferred_element_type=jnp.float32)
                + be_ref[0])


def _expert_mm(disp, We, be3):
    return pl.pallas_call(
        _expert_body,
        grid=(E,),
        in_specs=[
            pl.BlockSpec((1, CAP, D), lambda e: (e, 0, 0)),
            pl.BlockSpec((1, D, D), lambda e: (e, 0, 0)),
            pl.BlockSpec((1, 1, D), lambda e: (e, 0, 0)),
        ],
        out_specs=pl.BlockSpec((1, CAP, D), lambda e: (e, 0, 0)),
        out_shape=jax.ShapeDtypeStruct((E, CAP, D), jnp.float32),
    )(disp, We, be3)


def _combine_body(y_hbm, d0_hbm, d1_hbm, g0_hbm, g1_hbm, out_hbm,
                  d0_v, d1_v, g0_v, g1_v, b0_v, b1_v, ob_v, sem0, sem1):
    wid = lax.axis_index("s") * NC + lax.axis_index("c")
    base = wid * TPT

    pltpu.sync_copy(d0_hbm.at[pl.ds(base, TPT)], d0_v)
    pltpu.sync_copy(d1_hbm.at[pl.ds(base, TPT)], d1_v)
    pltpu.sync_copy(g0_hbm.at[pl.ds(base, TPT)], g0_v)
    pltpu.sync_copy(g1_hbm.at[pl.ds(base, TPT)], g1_v)

    def chunk_body(ch, _):
        t0 = ch * CCH
        cp0 = pltpu.async_copy(y_hbm.at[d0_v.at[pl.ds(t0, CCH)]], b0_v, sem0)
        cp1 = pltpu.async_copy(y_hbm.at[d1_v.at[pl.ds(t0, CCH)]], b1_v, sem1)
        cp0.wait()
        cp1.wait()

        def tok_body(t, _):
            bcast = jnp.zeros((L,), jnp.int32) + (t0 + t)
            gt0 = plsc.load_gather(g0_v, [bcast])
            gt1 = plsc.load_gather(g1_v, [bcast])
            for j in range(D // L):
                sl = pl.ds(j * L, L)
                ob_v[t, sl] = b0_v[t, sl] * gt0 + b1_v[t, sl] * gt1
            return 0
        lax.fori_loop(0, CCH, tok_body, 0)
        pltpu.sync_copy(ob_v, out_hbm.at[pl.ds(base + t0, CCH)])
        return 0
    lax.fori_loop(0, TPT // CCH, chunk_body, 0)


def _combine(y, d0, d1, g0, g1):
    mesh = plsc.VectorSubcoreMesh(core_axis_name="c", subcore_axis_name="s")
    f = functools.partial(
        pl.kernel,
        mesh=mesh,
        compiler_params=pltpu.CompilerParams(needs_layout_passes=False),
        out_type=jax.ShapeDtypeStruct((N, D), jnp.float32),
        scratch_types=[
            pltpu.VMEM((TPT,), jnp.int32),
            pltpu.VMEM((TPT,), jnp.int32),
            pltpu.VMEM((TPT,), jnp.float32),
            pltpu.VMEM((TPT,), jnp.float32),
            pltpu.VMEM((CCH, D), jnp.float32),
            pltpu.VMEM((CCH, D), jnp.float32),
            pltpu.VMEM((CCH, D), jnp.float32),
            pltpu.SemaphoreType.DMA,
            pltpu.SemaphoreType.DMA,
        ],
    )(_combine_body)
    return f(y, d0, d1, g0, g1)


def kernel(hidden_states, Wg, We, be):
    x = hidden_states
    d0, d1, s0, s1, g0, g1 = _router(x, Wg)
    d0 = d0.reshape(N)
    d1 = d1.reshape(N)
    s0 = s0.reshape(N)
    s1 = s1.reshape(N)
    g0 = g0.reshape(N)
    g1 = g1.reshape(N)

    x_pad = jnp.concatenate([x, jnp.zeros((8, D), jnp.float32)], axis=0)
    disp = _dispatch(x_pad, s0, s1)
    y = _expert_mm(disp.reshape(E, CAP, D), We, be.reshape(E, 1, D))
    return _combine(y.reshape(NSLOT, D), d0, d1, g0, g1)


# speedup vs baseline: 2.9974x; 1.5568x over previous
import functools

import jax
import jax.numpy as jnp
from jax import lax
from jax.experimental import pallas as pl
from jax.experimental.pallas import tpu as pltpu
from jax.experimental.pallas import tpu_sc as plsc

E = 64
K = 2
D = 768
N = 4096
CAP = 160
NSLOT = E * CAP
TRASH = NSLOT
PAD_ROW = N

B = 512
NB = N // B

NC = 2
NS = 16
NW = NC * NS
L = 16

TPT = N // NW
CCH = 16


def _router_body(x_ref, wg_ref, d0_ref, d1_ref, s0_ref, s1_ref,
                 g0_ref, g1_ref, carry_ref):
    i = pl.program_id(0)

    @pl.when(i == 0)
    def _():
        carry_ref[...] = jnp.zeros_like(carry_ref)

    x = x_ref[...]
    wg = wg_ref[...]
    logits = jnp.dot(x, wg, preferred_element_type=jnp.float32)

    iota = lax.broadcasted_iota(jnp.int32, (B, E), 1)
    m0 = jnp.max(logits, axis=1, keepdims=True)
    idx0 = jnp.min(jnp.where(logits == m0, iota, E), axis=1, keepdims=True)
    oh0 = iota == idx0
    masked = jnp.where(oh0, -jnp.inf, logits)
    m1 = jnp.max(masked, axis=1, keepdims=True)
    idx1 = jnp.min(jnp.where(masked == m1, iota, E), axis=1, keepdims=True)
    oh1 = iota == idx1

    t = jnp.exp(m1 - m0)
    g0 = 1.0 / (1.0 + t)
    g1 = t / (1.0 + t)

    ohsum = jnp.where(oh0 | oh1, 1.0, 0.0)
    r = lax.broadcasted_iota(jnp.int32, (B, B), 0)
    c = lax.broadcasted_iota(jnp.int32, (B, B), 1)
    tril = jnp.where(c < r, 1.0, 0.0)
    cnt_before = (jnp.dot(tril, ohsum, preferred_element_type=jnp.float32)
                  + carry_ref[...])
    carry_ref[...] = carry_ref[...] + jnp.sum(ohsum, axis=0, keepdims=True)

    pos0 = jnp.sum(jnp.where(oh0, cnt_before, 0.0), axis=1)
    pos1 = jnp.sum(jnp.where(oh1, cnt_before, 0.0), axis=1)
    keep0 = pos0 < CAP
    keep1 = pos1 < CAP
    p0c = jnp.minimum(pos0.astype(jnp.int32), CAP - 1)
    p1c = jnp.minimum(pos1.astype(jnp.int32), CAP - 1)
    e0 = idx0[:, 0]
    e1 = idx1[:, 0]
    d0 = e0 * CAP + p0c
    d1 = e1 * CAP + p1c

    d0_ref[...] = d0.reshape(1, 1, B)
    d1_ref[...] = d1.reshape(1, 1, B)
    s0_ref[...] = jnp.where(keep0, d0, TRASH).reshape(1, 1, B)
    s1_ref[...] = jnp.where(keep1, d1, TRASH).reshape(1, 1, B)
    g0_ref[...] = (g0[:, 0] * keep0.astype(jnp.float32)).reshape(1, 1, B)
    g1_ref[...] = (g1[:, 0] * keep1.astype(jnp.float32)).reshape(1, 1, B)


def _router(x, Wg):
    blk = pl.BlockSpec((1, 1, B), lambda i: (i, 0, 0))
    iod = jax.ShapeDtypeStruct((NB, 1, B), jnp.int32)
    fod = jax.ShapeDtypeStruct((NB, 1, B), jnp.float32)
    return pl.pallas_call(
        _router_body,
        grid=(NB,),
        in_specs=[
            pl.BlockSpec((B, D), lambda i: (i, 0)),
            pl.BlockSpec((D, E), lambda i: (0, 0)),
        ],
        out_specs=[blk] * 6,
        out_shape=[iod, iod, iod, iod, fod, fod],
        scratch_shapes=[pltpu.VMEM((1, E), jnp.float32)],
    )(x, Wg)


DCH = 64
DNCH = (N // NW) // DCH


def _dispatch_body(x_hbm, s0_hbm, s1_hbm, disp_hbm,
                   idx_v, xb0_v, xb1_v, sem_in, sem_out):
    wid = lax.axis_index("s") * NC + lax.axis_index("c")
    base = wid * (N // NW)

    for ch in range(DNCH):
        t0 = base + ch * DCH
        pltpu.sync_copy(s0_hbm.at[pl.ds(t0, DCH)], idx_v.at[2 * ch])
        pltpu.sync_copy(s1_hbm.at[pl.ds(t0, DCH)], idx_v.at[2 * ch + 1])

    bufs = [xb0_v, xb1_v]
    cp = pltpu.async_copy(x_hbm.at[pl.ds(base, DCH)], bufs[0], sem_in)
    cp.wait()
    for ch in range(DNCH):
        buf = bufs[ch % 2]
        if ch + 1 < DNCH:
            nxt = pltpu.async_copy(
                x_hbm.at[pl.ds(base + (ch + 1) * DCH, DCH)],
                bufs[(ch + 1) % 2], sem_in)
        o0 = pltpu.async_copy(buf, disp_hbm.at[idx_v.at[2 * ch]], sem_out)
        o1 = pltpu.async_copy(buf, disp_hbm.at[idx_v.at[2 * ch + 1]], sem_out)
        o0.wait()
        o1.wait()
        if ch + 1 < DNCH:
            nxt.wait()


def _dispatch(x, s0, s1):
    mesh = plsc.VectorSubcoreMesh(core_axis_name="c", subcore_axis_name="s")
    f = functools.partial(
        pl.kernel,
        mesh=mesh,
        compiler_params=pltpu.CompilerParams(needs_layout_passes=False),
        out_type=jax.ShapeDtypeStruct((NSLOT + 8, D), jnp.float32),
        scratch_types=[
            pltpu.VMEM((2 * DNCH, DCH), jnp.int32),
            pltpu.VMEM((DCH, D), jnp.float32),
            pltpu.VMEM((DCH, D), jnp.float32),
            pltpu.SemaphoreType.DMA,
            pltpu.SemaphoreType.DMA,
        ],
    )(_dispatch_body)
    return f(x, s0, s1)


def _expert_body(disp_ref, we_ref, be_ref, y_ref):
    a = disp_ref[0]
    w = we_ref[0]
    y_ref[0] = (jnp.dot(a, w, preferred_element_type=jnp.float32)
                + be_ref[0])


def _expert_mm(disp, We, be3):
    return pl.pallas_call(
        _expert_body,
        grid=(E,),
        in_specs=[
            pl.BlockSpec((1, CAP, D), lambda e: (e, 0, 0)),
            pl.BlockSpec((1, D, D), lambda e: (e, 0, 0)),
            pl.BlockSpec((1, 1, D), lambda e: (e, 0, 0)),
        ],
        out_specs=pl.BlockSpec((1, CAP, D), lambda e: (e, 0, 0)),
        out_shape=jax.ShapeDtypeStruct((E, CAP, D), jnp.float32),
    )(disp, We, be3)


def _combine_body(y_hbm, d0_hbm, d1_hbm, g0_hbm, g1_hbm, out_hbm,
                  d0_v, d1_v, g0_v, g1_v, b0_v, b1_v, ob_v, sem0, sem1):
    wid = lax.axis_index("s") * NC + lax.axis_index("c")
    base = wid * TPT

    pltpu.sync_copy(d0_hbm.at[pl.ds(base, TPT)], d0_v)
    pltpu.sync_copy(d1_hbm.at[pl.ds(base, TPT)], d1_v)
    pltpu.sync_copy(g0_hbm.at[pl.ds(base, TPT)], g0_v)
    pltpu.sync_copy(g1_hbm.at[pl.ds(base, TPT)], g1_v)

    def chunk_body(ch, _):
        t0 = ch * CCH
        cp0 = pltpu.async_copy(y_hbm.at[d0_v.at[pl.ds(t0, CCH)]], b0_v, sem0)
        cp1 = pltpu.async_copy(y_hbm.at[d1_v.at[pl.ds(t0, CCH)]], b1_v, sem1)
        cp0.wait()
        cp1.wait()

        def tok_body(t, _):
            bcast = jnp.zeros((L,), jnp.int32) + (t0 + t)
            gt0 = plsc.load_gather(g0_v, [bcast])
            gt1 = plsc.load_gather(g1_v, [bcast])
            for j in range(D // L):
                sl = pl.ds(j * L, L)
                ob_v[t, sl] = b0_v[t, sl] * gt0 + b1_v[t, sl] * gt1
            return 0
        lax.fori_loop(0, CCH, tok_body, 0)
        pltpu.sync_copy(ob_v, out_hbm.at[pl.ds(base + t0, CCH)])
        return 0
    lax.fori_loop(0, TPT // CCH, chunk_body, 0)


def _combine(y, d0, d1, g0, g1):
    mesh = plsc.VectorSubcoreMesh(core_axis_name="c", subcore_axis_name="s")
    f = functools.partial(
        pl.kernel,
        mesh=mesh,
        compiler_params=pltpu.CompilerParams(needs_layout_passes=False),
        out_type=jax.ShapeDtypeStruct((N, D), jnp.float32),
        scratch_types=[
            pltpu.VMEM((TPT,), jnp.int32),
            pltpu.VMEM((TPT,), jnp.int32),
            pltpu.VMEM((TPT,), jnp.float32),
            pltpu.VMEM((TPT,), jnp.float32),
            pltpu.VMEM((CCH, D), jnp.float32),
            pltpu.VMEM((CCH, D), jnp.float32),
            pltpu.VMEM((CCH, D), jnp.float32),
            pltpu.SemaphoreType.DMA,
            pltpu.SemaphoreType.DMA,
        ],
    )(_combine_body)
    return f(y, d0, d1, g0, g1)


def kernel(hidden_states, Wg, We, be):
    x = hidden_states
    d0, d1, s0, s1, g0, g1 = _router(x, Wg)
    d0 = d0.reshape(N)
    d1 = d1.reshape(N)
    s0 = s0.reshape(N)
    s1 = s1.reshape(N)
    g0 = g0.reshape(N)
    g1 = g1.reshape(N)

    disp = _dispatch(x, s0, s1)[:NSLOT]
    y = _expert_mm(disp.reshape(E, CAP, D), We, be.reshape(E, 1, D))
    return _combine(y.reshape(NSLOT, D), d0, d1, g0, g1)


# speedup vs baseline: 3.0315x; 1.0114x over previous
import functools

import jax
import jax.numpy as jnp
from jax import lax
from jax.experimental import pallas as pl
from jax.experimental.pallas import tpu as pltpu
from jax.experimental.pallas import tpu_sc as plsc

E = 64
K = 2
D = 768
N = 4096
CAP = 160
NSLOT = E * CAP
TRASH = NSLOT
PAD_ROW = N

B = 512
NB = N // B

NC = 2
NS = 16
NW = NC * NS
L = 16

TPT = N // NW
CCH = 16


def _router_body(x_ref, wg_ref, d0_ref, d1_ref, s0_ref, s1_ref,
                 g0_ref, g1_ref, carry_ref):
    i = pl.program_id(0)

    @pl.when(i == 0)
    def _():
        carry_ref[...] = jnp.zeros_like(carry_ref)

    x = x_ref[...]
    wg = wg_ref[...]
    logits = jnp.dot(x, wg, preferred_element_type=jnp.float32)

    iota = lax.broadcasted_iota(jnp.int32, (B, E), 1)
    m0 = jnp.max(logits, axis=1, keepdims=True)
    idx0 = jnp.min(jnp.where(logits == m0, iota, E), axis=1, keepdims=True)
    oh0 = iota == idx0
    masked = jnp.where(oh0, -jnp.inf, logits)
    m1 = jnp.max(masked, axis=1, keepdims=True)
    idx1 = jnp.min(jnp.where(masked == m1, iota, E), axis=1, keepdims=True)
    oh1 = iota == idx1

    t = jnp.exp(m1 - m0)
    g0 = 1.0 / (1.0 + t)
    g1 = t / (1.0 + t)

    ohsum = jnp.where(oh0 | oh1, 1.0, 0.0)
    r = lax.broadcasted_iota(jnp.int32, (B, B), 0)
    c = lax.broadcasted_iota(jnp.int32, (B, B), 1)
    tril = jnp.where(c < r, 1.0, 0.0)
    cnt_before = (jnp.dot(tril, ohsum, preferred_element_type=jnp.float32)
                  + carry_ref[...])
    carry_ref[...] = carry_ref[...] + jnp.sum(ohsum, axis=0, keepdims=True)

    pos0 = jnp.sum(jnp.where(oh0, cnt_before, 0.0), axis=1)
    pos1 = jnp.sum(jnp.where(oh1, cnt_before, 0.0), axis=1)
    keep0 = pos0 < CAP
    keep1 = pos1 < CAP
    p0c = jnp.minimum(pos0.astype(jnp.int32), CAP - 1)
    p1c = jnp.minimum(pos1.astype(jnp.int32), CAP - 1)
    e0 = idx0[:, 0]
    e1 = idx1[:, 0]
    d0 = e0 * CAP + p0c
    d1 = e1 * CAP + p1c

    d0_ref[...] = d0.reshape(1, 1, B)
    d1_ref[...] = d1.reshape(1, 1, B)
    s0_ref[...] = jnp.where(keep0, d0, TRASH).reshape(1, 1, B)
    s1_ref[...] = jnp.where(keep1, d1, TRASH).reshape(1, 1, B)
    g0_ref[...] = (g0[:, 0] * keep0.astype(jnp.float32)).reshape(1, 1, B)
    g1_ref[...] = (g1[:, 0] * keep1.astype(jnp.float32)).reshape(1, 1, B)


def _router(x, Wg):
    blk = pl.BlockSpec((1, 1, B), lambda i: (i, 0, 0))
    iod = jax.ShapeDtypeStruct((NB, 1, B), jnp.int32)
    fod = jax.ShapeDtypeStruct((NB, 1, B), jnp.float32)
    return pl.pallas_call(
        _router_body,
        grid=(NB,),
        in_specs=[
            pl.BlockSpec((B, D), lambda i: (i, 0)),
            pl.BlockSpec((D, E), lambda i: (0, 0)),
        ],
        out_specs=[blk] * 6,
        out_shape=[iod, iod, iod, iod, fod, fod],
        scratch_shapes=[pltpu.VMEM((1, E), jnp.float32)],
    )(x, Wg)


DCH = 64
DNCH = (N // NW) // DCH


def _dispatch_body(x_hbm, s0_hbm, s1_hbm, disp_hbm,
                   idx_v, xb0_v, xb1_v, sem_in, sem_out):
    wid = lax.axis_index("s") * NC + lax.axis_index("c")
    base = wid * (N // NW)

    for ch in range(DNCH):
        t0 = base + ch * DCH
        pltpu.sync_copy(s0_hbm.at[pl.ds(t0, DCH)], idx_v.at[2 * ch])
        pltpu.sync_copy(s1_hbm.at[pl.ds(t0, DCH)], idx_v.at[2 * ch + 1])

    bufs = [xb0_v, xb1_v]
    cp = pltpu.async_copy(x_hbm.at[pl.ds(base, DCH)], bufs[0], sem_in)
    cp.wait()
    for ch in range(DNCH):
        buf = bufs[ch % 2]
        if ch + 1 < DNCH:
            nxt = pltpu.async_copy(
                x_hbm.at[pl.ds(base + (ch + 1) * DCH, DCH)],
                bufs[(ch + 1) % 2], sem_in)
        o0 = pltpu.async_copy(buf, disp_hbm.at[idx_v.at[2 * ch]], sem_out)
        o1 = pltpu.async_copy(buf, disp_hbm.at[idx_v.at[2 * ch + 1]], sem_out)
        o0.wait()
        o1.wait()
        if ch + 1 < DNCH:
            nxt.wait()


def _dispatch(x, s0, s1):
    mesh = plsc.VectorSubcoreMesh(core_axis_name="c", subcore_axis_name="s")
    f = functools.partial(
        pl.kernel,
        mesh=mesh,
        compiler_params=pltpu.CompilerParams(needs_layout_passes=False),
        out_type=jax.ShapeDtypeStruct((NSLOT + 8, D), jnp.float32),
        scratch_types=[
            pltpu.VMEM((2 * DNCH, DCH), jnp.int32),
            pltpu.VMEM((DCH, D), jnp.float32),
            pltpu.VMEM((DCH, D), jnp.float32),
            pltpu.SemaphoreType.DMA,
            pltpu.SemaphoreType.DMA,
        ],
    )(_dispatch_body)
    return f(x, s0, s1)


def _expert_body(disp_ref, we_ref, be_ref, y_ref):
    a = disp_ref[0]
    w = we_ref[0]
    y_ref[0] = (jnp.dot(a, w, preferred_element_type=jnp.float32)
                + be_ref[0])


def _expert_mm(disp, We, be3):
    return pl.pallas_call(
        _expert_body,
        grid=(E,),
        in_specs=[
            pl.BlockSpec((1, CAP, D), lambda e: (e, 0, 0)),
            pl.BlockSpec((1, D, D), lambda e: (e, 0, 0)),
            pl.BlockSpec((1, 1, D), lambda e: (e, 0, 0)),
        ],
        out_specs=pl.BlockSpec((1, CAP, D), lambda e: (e, 0, 0)),
        out_shape=jax.ShapeDtypeStruct((E, CAP, D), jnp.float32),
    )(disp, We, be3)


def _combine_body(y_hbm, d0_hbm, d1_hbm, g0_hbm, g1_hbm, out_hbm,
                  d0_v, d1_v, g0_v, g1_v,
                  b0a_v, b1a_v, b0b_v, b1b_v, oba_v, obb_v,
                  sga, sgb, sw):
    wid = lax.axis_index("s") * NC + lax.axis_index("c")
    base = wid * TPT
    nch = TPT // CCH

    pltpu.sync_copy(d0_hbm.at[pl.ds(base, TPT)], d0_v)
    pltpu.sync_copy(d1_hbm.at[pl.ds(base, TPT)], d1_v)
    pltpu.sync_copy(g0_hbm.at[pl.ds(base, TPT)], g0_v)
    pltpu.sync_copy(g1_hbm.at[pl.ds(base, TPT)], g1_v)

    b0s = [b0a_v, b0b_v]
    b1s = [b1a_v, b1b_v]
    obs = [oba_v, obb_v]
    sgs = [sga, sgb]

    def gathers(ch, k):
        t0 = ch * CCH
        pltpu.async_copy(y_hbm.at[d0_v.at[pl.ds(t0, CCH)]], b0s[k], sgs[k])
        pltpu.async_copy(y_hbm.at[d1_v.at[pl.ds(t0, CCH)]], b1s[k], sgs[k])

    gathers(0, 0)
    for ch in range(nch):
        k = ch % 2
        if ch + 1 < nch:
            gathers(ch + 1, 1 - k)
        pltpu.make_async_copy(y_hbm.at[d0_v.at[pl.ds(0, CCH)]],
                              b0s[k], sgs[k]).wait()
        pltpu.make_async_copy(y_hbm.at[d1_v.at[pl.ds(0, CCH)]],
                              b1s[k], sgs[k]).wait()
        if ch >= 2:
            pltpu.make_async_copy(obs[k], out_hbm.at[pl.ds(0, CCH)],
                                  sw).wait()
        t0 = ch * CCH

        def tok_body(t, _):
            bcast = jnp.zeros((L,), jnp.int32) + (t0 + t)
            gt0 = plsc.load_gather(g0_v, [bcast])
            gt1 = plsc.load_gather(g1_v, [bcast])
            for j in range(D // L):
                sl = pl.ds(j * L, L)
                obs[k][t, sl] = b0s[k][t, sl] * gt0 + b1s[k][t, sl] * gt1
            return 0
        lax.fori_loop(0, CCH, tok_body, 0)
        pltpu.async_copy(obs[k], out_hbm.at[pl.ds(base + t0, CCH)], sw)
    pltpu.make_async_copy(obs[0], out_hbm.at[pl.ds(0, CCH)], sw).wait()
    pltpu.make_async_copy(obs[0], out_hbm.at[pl.ds(0, CCH)], sw).wait()


def _combine(y, d0, d1, g0, g1):
    mesh = plsc.VectorSubcoreMesh(core_axis_name="c", subcore_axis_name="s")
    f = functools.partial(
        pl.kernel,
        mesh=mesh,
        compiler_params=pltpu.CompilerParams(needs_layout_passes=False),
        out_type=jax.ShapeDtypeStruct((N, D), jnp.float32),
        scratch_types=[
            pltpu.VMEM((TPT,), jnp.int32),
            pltpu.VMEM((TPT,), jnp.int32),
            pltpu.VMEM((TPT,), jnp.float32),
            pltpu.VMEM((TPT,), jnp.float32),
            pltpu.VMEM((CCH, D), jnp.float32),
            pltpu.VMEM((CCH, D), jnp.float32),
            pltpu.VMEM((CCH, D), jnp.float32),
            pltpu.VMEM((CCH, D), jnp.float32),
            pltpu.VMEM((CCH, D), jnp.float32),
            pltpu.VMEM((CCH, D), jnp.float32),
            pltpu.SemaphoreType.DMA,
            pltpu.SemaphoreType.DMA,
            pltpu.SemaphoreType.DMA,
        ],
    )(_combine_body)
    return f(y, d0, d1, g0, g1)


def kernel(hidden_states, Wg, We, be):
    x = hidden_states
    d0, d1, s0, s1, g0, g1 = _router(x, Wg)
    d0 = d0.reshape(N)
    d1 = d1.reshape(N)
    s0 = s0.reshape(N)
    s1 = s1.reshape(N)
    g0 = g0.reshape(N)
    g1 = g1.reshape(N)

    disp = _dispatch(x, s0, s1)[:NSLOT]
    y = _expert_mm(disp.reshape(E, CAP, D), We, be.reshape(E, 1, D))
    return _combine(y.reshape(NSLOT, D), d0, d1, g0, g1)
